# Initial kernel scaffold; baseline (speedup 1.0000x reference)
import jax
import jax.numpy as jnp
from jax.experimental import pallas as pl


def kernel(input_ids, embed_tokens):
    raise NotImplementedError("write your pallas kernel here")


# speedup vs baseline: 1.5696x; 1.5696x over previous
import functools

import jax
import jax.numpy as jnp
from jax import lax
from jax.experimental import pallas as pl
from jax.experimental.pallas import tpu as pltpu
from jax.experimental.pallas import tpu_sc as plsc

_NUM_WORKERS = 32
_CHUNK = 64


@functools.lru_cache(maxsize=None)
def _build_gather(V, D, n_total):
    n_per_w = n_total // _NUM_WORKERS
    n_chunks = n_per_w // _CHUNK
    mesh = plsc.VectorSubcoreMesh(core_axis_name="c", subcore_axis_name="s")

    @functools.partial(
        pl.kernel,
        mesh=mesh,
        out_type=jax.ShapeDtypeStruct((n_total, D), jnp.float32),
        scratch_types=[
            pltpu.VMEM((n_chunks, _CHUNK), jnp.int32),
            pltpu.VMEM((_CHUNK, D), jnp.float32),
            pltpu.SemaphoreType.DMA,
        ],
    )
    def gather_kernel(table_hbm, idx_hbm, out_hbm, idx_v, rows_v, sem):
        wid = lax.axis_index("s") * 2 + lax.axis_index("c")
        pltpu.sync_copy(idx_hbm.at[wid], idx_v)
        base = wid * n_per_w

        def body(j, carry):
            pltpu.async_copy(table_hbm.at[idx_v.at[j]], rows_v, sem).wait()
            pltpu.sync_copy(rows_v, out_hbm.at[pl.ds(base + j * _CHUNK, _CHUNK)])
            return carry

        lax.fori_loop(0, n_chunks, body, 0)

    return gather_kernel


def kernel(input_ids, embed_tokens):
    B, S = input_ids.shape
    V, D = embed_tokens.shape
    n_total = B * S
    ids = input_ids.reshape(_NUM_WORKERS, (n_total // _NUM_WORKERS) // _CHUNK,
                            _CHUNK).astype(jnp.int32)
    out = _build_gather(V, D, n_total)(embed_tokens, ids)
    return out.reshape(B, S, D)


# speedup vs baseline: 1.6694x; 1.0636x over previous
import functools

import jax
import jax.numpy as jnp
from jax import lax
from jax.experimental import pallas as pl
from jax.experimental.pallas import tpu as pltpu
from jax.experimental.pallas import tpu_sc as plsc

_NUM_WORKERS = 32
_CHUNK = 32


@functools.lru_cache(maxsize=None)
def _build_gather(V, D, n_total):
    n_per_w = n_total // _NUM_WORKERS
    n_chunks = n_per_w // _CHUNK
    mesh = plsc.VectorSubcoreMesh(core_axis_name="c", subcore_axis_name="s")

    @functools.partial(
        pl.kernel,
        mesh=mesh,
        out_type=jax.ShapeDtypeStruct((n_total, D), jnp.float32),
        scratch_types=[
            pltpu.VMEM((n_chunks, _CHUNK), jnp.int32),
            pltpu.VMEM((_CHUNK, D), jnp.float32),
            pltpu.VMEM((_CHUNK, D), jnp.float32),
            pltpu.SemaphoreType.DMA,
            pltpu.SemaphoreType.DMA,
        ],
    )
    def gather_kernel(table_hbm, idx_hbm, out_hbm, idx_v, rows0, rows1,
                      sem0, sem1):
        wid = lax.axis_index("s") * 2 + lax.axis_index("c")
        pltpu.sync_copy(idx_hbm.at[wid], idx_v)
        base = wid * n_per_w
        bufs = ((rows0, sem0), (rows1, sem1))

        pltpu.async_copy(table_hbm.at[idx_v.at[0]], rows0, sem0)
        pltpu.async_copy(table_hbm.at[idx_v.at[1]], rows1, sem1)

        def body(i, carry):
            for b, (buf, sem) in enumerate(bufs):
                j = 2 * i + b
                pltpu.make_async_copy(table_hbm.at[idx_v.at[j]], buf, sem).wait()
                pltpu.sync_copy(buf, out_hbm.at[pl.ds(base + j * _CHUNK, _CHUNK)])
                pltpu.async_copy(table_hbm.at[idx_v.at[j + 2]], buf, sem)
            return carry

        lax.fori_loop(0, n_chunks // 2 - 1, body, 0)

        for b, (buf, sem) in enumerate(bufs):
            j = n_chunks - 2 + b
            pltpu.make_async_copy(table_hbm.at[idx_v.at[j]], buf, sem).wait()
            pltpu.sync_copy(buf, out_hbm.at[pl.ds(base + j * _CHUNK, _CHUNK)])

    return gather_kernel


def kernel(input_ids, embed_tokens):
    B, S = input_ids.shape
    V, D = embed_tokens.shape
    n_total = B * S
    ids = input_ids.reshape(_NUM_WORKERS, (n_total // _NUM_WORKERS) // _CHUNK,
                            _CHUNK).astype(jnp.int32)
    out = _build_gather(V, D, n_total)(embed_tokens, ids)
    return out.reshape(B, S, D)


# speedup vs baseline: 1.6712x; 1.0011x over previous
import functools

import jax
import jax.numpy as jnp
from jax import lax
from jax.experimental import pallas as pl
from jax.experimental.pallas import tpu as pltpu
from jax.experimental.pallas import tpu_sc as plsc

_NUM_WORKERS = 32
_CHUNK = 16


@functools.lru_cache(maxsize=None)
def _build_gather(V, D, n_total):
    n_per_w = n_total // _NUM_WORKERS
    n_chunks = n_per_w // _CHUNK
    mesh = plsc.VectorSubcoreMesh(core_axis_name="c", subcore_axis_name="s")

    @functools.partial(
        pl.kernel,
        mesh=mesh,
        out_type=jax.ShapeDtypeStruct((n_total, D), jnp.float32),
        scratch_types=[
            pltpu.VMEM((n_chunks, _CHUNK), jnp.int32),
            pltpu.VMEM((_CHUNK, D), jnp.float32),
            pltpu.VMEM((_CHUNK, D), jnp.float32),
            pltpu.VMEM((_CHUNK, D), jnp.float32),
            pltpu.VMEM((_CHUNK, D), jnp.float32),
            pltpu.SemaphoreType.DMA,
            pltpu.SemaphoreType.DMA,
            pltpu.SemaphoreType.DMA,
            pltpu.SemaphoreType.DMA,
        ],
    )
    def gather_kernel(table_hbm, idx_hbm, out_hbm, idx_v, rows0, rows1, rows2,
                      rows3, sem0, sem1, sem2, sem3):
        wid = lax.axis_index("s") * 2 + lax.axis_index("c")
        pltpu.sync_copy(idx_hbm.at[wid], idx_v)
        base = wid * n_per_w
        nbuf = 4
        bufs = ((rows0, sem0), (rows1, sem1), (rows2, sem2), (rows3, sem3))

        for b, (buf, sem) in enumerate(bufs):
            pltpu.async_copy(table_hbm.at[idx_v.at[b]], buf, sem)

        def body(i, carry):
            for b, (buf, sem) in enumerate(bufs):
                j = nbuf * i + b
                pltpu.make_async_copy(table_hbm.at[idx_v.at[j]], buf, sem).wait()
                pltpu.sync_copy(buf, out_hbm.at[pl.ds(base + j * _CHUNK, _CHUNK)])
                pltpu.async_copy(table_hbm.at[idx_v.at[j + nbuf]], buf, sem)
            return carry

        lax.fori_loop(0, n_chunks // nbuf - 1, body, 0)

        for b, (buf, sem) in enumerate(bufs):
            j = n_chunks - nbuf + b
            pltpu.make_async_copy(table_hbm.at[idx_v.at[j]], buf, sem).wait()
            pltpu.sync_copy(buf, out_hbm.at[pl.ds(base + j * _CHUNK, _CHUNK)])

    return gather_kernel


def kernel(input_ids, embed_tokens):
    B, S = input_ids.shape
    V, D = embed_tokens.shape
    n_total = B * S
    ids = input_ids.reshape(_NUM_WORKERS, (n_total // _NUM_WORKERS) // _CHUNK,
                            _CHUNK).astype(jnp.int32)
    out = _build_gather(V, D, n_total)(embed_tokens, ids)
    return out.reshape(B, S, D)


# speedup vs baseline: 1.6875x; 1.0098x over previous
import functools

import jax
import jax.numpy as jnp
from jax import lax
from jax.experimental import pallas as pl
from jax.experimental.pallas import tpu as pltpu
from jax.experimental.pallas import tpu_sc as plsc

_NUM_WORKERS = 32
_CHUNK = 8
_NBUF = 8


@functools.lru_cache(maxsize=None)
def _build_gather(V, D, n_total):
    n_per_w = n_total // _NUM_WORKERS
    n_chunks = n_per_w // _CHUNK
    mesh = plsc.VectorSubcoreMesh(core_axis_name="c", subcore_axis_name="s")

    @functools.partial(
        pl.kernel,
        mesh=mesh,
        out_type=jax.ShapeDtypeStruct((n_total, D), jnp.float32),
        scratch_types=(
            [pltpu.VMEM((n_chunks, _CHUNK), jnp.int32)]
            + [pltpu.VMEM((_CHUNK, D), jnp.float32)] * _NBUF
            + [pltpu.SemaphoreType.DMA] * _NBUF
        ),
    )
    def gather_kernel(table_hbm, idx_hbm, out_hbm, idx_v, *rest):
        bufs = tuple(zip(rest[:_NBUF], rest[_NBUF:]))
        wid = lax.axis_index("s") * 2 + lax.axis_index("c")
        pltpu.sync_copy(idx_hbm.at[wid], idx_v)
        base = wid * n_per_w

        for b, (buf, sem) in enumerate(bufs):
            pltpu.async_copy(table_hbm.at[idx_v.at[b]], buf, sem)

        def body(i, carry):
            for b, (buf, sem) in enumerate(bufs):
                j = _NBUF * i + b
                pltpu.make_async_copy(table_hbm.at[idx_v.at[j]], buf, sem).wait()
                pltpu.sync_copy(buf, out_hbm.at[pl.ds(base + j * _CHUNK, _CHUNK)])
                pltpu.async_copy(table_hbm.at[idx_v.at[j + _NBUF]], buf, sem)
            return carry

        lax.fori_loop(0, n_chunks // _NBUF - 1, body, 0)

        for b, (buf, sem) in enumerate(bufs):
            j = n_chunks - _NBUF + b
            pltpu.make_async_copy(table_hbm.at[idx_v.at[j]], buf, sem).wait()
            pltpu.sync_copy(buf, out_hbm.at[pl.ds(base + j * _CHUNK, _CHUNK)])

    return gather_kernel


def kernel(input_ids, embed_tokens):
    B, S = input_ids.shape
    V, D = embed_tokens.shape
    n_total = B * S
    ids = input_ids.reshape(_NUM_WORKERS, (n_total // _NUM_WORKERS) // _CHUNK,
                            _CHUNK).astype(jnp.int32)
    out = _build_gather(V, D, n_total)(embed_tokens, ids)
    return out.reshape(B, S, D)
